# baseline (speedup 1.0000x reference)
import functools

import jax
import jax.numpy as jnp
from jax import lax
from jax.experimental import pallas as pl
from jax.experimental.pallas import tpu as pltpu
from jax.experimental.pallas import tpu_sc as plsc

N = 2048
E = 131072
D_IN = 64
DE_IN = 16
CNODE = 64
CEDGE = 32
MSG = 64
CONV = 64
CLNK = 16
NCLS = 5
NECLS = 2
ROFF = 2
K = 128
M = 32

LW = 128

NC = 2
NS = 16
NW = NC * NS
CH = 128
G = 512
EW = E // NW

_f32 = jnp.float32


def _sds(shape):
    return jax.ShapeDtypeStruct(shape, _f32)


def _mesh():
    return plsc.VectorSubcoreMesh(core_axis_name="c", subcore_axis_name="s",
                                  num_cores=NC, num_subcores=NS)


def _sc_gather_uv(tu, tv, src2, dst2):
    GG = 256

    @functools.partial(
        pl.kernel,
        out_type=[_sds((E, LW)), _sds((E, LW))],
        mesh=_mesh(),
        scratch_types=[
            pltpu.VMEM((EW // CH, CH), jnp.int32),
            pltpu.VMEM((EW // CH, CH), jnp.int32),
            pltpu.VMEM((GG, LW), _f32),
            pltpu.VMEM((GG, LW), _f32),
            pltpu.SemaphoreType.DMA,
        ],
    )
    def k(tu_hbm, tv_hbm, src_hbm, dst_hbm, gs_hbm, gd_hbm, ibs, ibd, ub, vb, sem):
        wid = lax.axis_index("s") * NC + lax.axis_index("c")
        row0 = wid * (EW // CH)
        pltpu.sync_copy(src_hbm.at[pl.ds(row0, EW // CH)], ibs)
        pltpu.sync_copy(dst_hbm.at[pl.ds(row0, EW // CH)], ibd)
        npc = GG // CH

        def body(g, carry):
            off = wid * EW + g * GG
            cps = []
            for j in range(npc):
                sl = pl.ds(j * CH, CH)
                cps.append(pltpu.async_copy(tu_hbm.at[ibs.at[g * npc + j]],
                                            ub.at[sl], sem))
                cps.append(pltpu.async_copy(tv_hbm.at[ibd.at[g * npc + j]],
                                            vb.at[sl], sem))
            for cp in cps:
                cp.wait()
            pltpu.sync_copy(ub, gs_hbm.at[pl.ds(off, GG)])
            pltpu.sync_copy(vb, gd_hbm.at[pl.ds(off, GG)])
            return carry

        lax.fori_loop(0, EW // GG, body, 0)

    return k(tu, tv, src2, dst2)


def _sc_scatter_add(m2, dst2, zer):
    @functools.partial(
        pl.kernel,
        out_type=_sds((NC, N, LW)),
        mesh=_mesh(),
        scratch_types=[
            pltpu.VMEM((EW // CH, CH), jnp.int32),
            pltpu.VMEM((G, LW), _f32),
            pltpu.VMEM_SHARED((N, LW), _f32),
        ],
    )
    def k(m_hbm, dst_hbm, zer_hbm, out_hbm, ib, mb, acc):
        c = lax.axis_index("c")
        s = lax.axis_index("s")

        @pl.when(s == 0)
        def _():
            pltpu.sync_copy(zer_hbm, acc)

        wid = s * NC + c
        row0 = wid * (EW // CH)
        pltpu.sync_copy(dst_hbm.at[pl.ds(row0, EW // CH)], ib)
        plsc.subcore_barrier()

        def body(g, carry):
            pltpu.sync_copy(m_hbm.at[pl.ds((row0 + g * (G // CH)) * CH, G)], mb)
            for j in range(G // CH):
                pltpu.sync_copy(mb.at[pl.ds(j * CH, CH)],
                                acc.at[ib.at[g * (G // CH) + j]], add=True)
            return carry

        lax.fori_loop(0, EW // G, body, 0)
        plsc.subcore_barrier()
        stripe = N // NS
        pltpu.sync_copy(acc.at[pl.ds(s * stripe, stripe)],
                        out_hbm.at[c, pl.ds(s * stripe, stripe)])

    return k(m2, dst2, zer)


def _sc_gather_rows(h128, cidx2):
    @functools.partial(
        pl.kernel,
        out_type=_sds((NW * CH, LW)),
        mesh=_mesh(),
        scratch_types=[
            pltpu.VMEM((CH,), jnp.int32),
            pltpu.VMEM((CH, LW), _f32),
            pltpu.SemaphoreType.DMA,
        ],
    )
    def k(h_hbm, ci_hbm, out_hbm, ib, hb, sem):
        wid = lax.axis_index("s") * NC + lax.axis_index("c")
        pltpu.sync_copy(ci_hbm.at[wid], ib)
        pltpu.async_copy(h_hbm.at[ib], hb, sem).wait()
        pltpu.sync_copy(hb, out_hbm.at[pl.ds(wid * CH, CH)])

    return k(h128, cidx2)


def _dot(a, b):
    return jax.lax.dot(a, b, preferred_element_type=_f32)


def _tc_node(nf, w1, b1, w2, b2, wa, wb):
    def k(nf_r, w1_r, b1_r, w2_r, b2_r, wa_r, wb_r, x_o, tu_o, tv_o):
        x = jnp.maximum(_dot(nf_r[...], w1_r[...]) + b1_r[...], 0.0)
        x = jnp.maximum(_dot(x, w2_r[...]) + b2_r[...], 0.0)
        x_o[...] = x
        z = jnp.zeros((N, LW - MSG), _f32)
        tu_o[...] = jnp.concatenate([_dot(x, wa_r[...]), z], axis=1)
        tv_o[...] = jnp.concatenate([_dot(x, wb_r[...]), z], axis=1)

    return pl.pallas_call(
        k, out_shape=[_sds((N, CNODE)), _sds((N, LW)), _sds((N, LW))],
    )(nf, w1, b1, w2, b2, wa, wb)


def _tc_edgemsg(ef, gs, gd, we1, be1, we2, be2, wc_m, bm1, wm2, bm2):
    BE = 8192
    grid = (E // BE,)

    def k(ef_r, gs_r, gd_r, we1_r, be1_r, we2_r, be2_r, wcm_r, bm1_r,
          wm2_r, bm2_r, m_o):
        e1 = jnp.maximum(_dot(ef_r[...], we1_r[...]) + be1_r[...], 0.0)
        e2 = jnp.maximum(_dot(e1, we2_r[...]) + be2_r[...], 0.0)
        t = _dot(e2, wcm_r[...]) + gs_r[:, :MSG] + gd_r[:, :MSG] + bm1_r[...]
        m1 = jnp.maximum(t, 0.0)
        m = jnp.maximum(_dot(m1, wm2_r[...]) + bm2_r[...], 0.0)
        m_o[...] = jnp.concatenate([m, jnp.zeros((BE, LW - MSG), _f32)], axis=1)

    def full(shape):
        return pl.BlockSpec(shape, lambda i: (0, 0))

    return pl.pallas_call(
        k,
        grid=grid,
        in_specs=[
            pl.BlockSpec((BE, DE_IN), lambda i: (i, 0)),
            pl.BlockSpec((BE, LW), lambda i: (i, 0)),
            pl.BlockSpec((BE, LW), lambda i: (i, 0)),
            full((DE_IN, CEDGE)), full((1, CEDGE)),
            full((CEDGE, CEDGE)), full((1, CEDGE)),
            full((CEDGE, MSG)), full((1, MSG)),
            full((MSG, MSG)), full((1, MSG)),
        ],
        out_specs=pl.BlockSpec((BE, LW), lambda i: (i, 0)),
        out_shape=_sds((E, LW)),
    )(ef, gs, gd, we1, be1, we2, be2, wc_m, bm1, wm2, bm2)


def _tc_update_heads(x, agg2, wux, wua, bu1, wu2, bu2,
                     wp1, bp1, wp2, bp2, wp3, bp3,
                     wo1, bo1, wo2, bo2, wo3, bo3, ws, wd, bsd):
    def k(x_r, agg_r, wux_r, wua_r, bu1_r, wu2_r, bu2_r,
          wp1_r, bp1_r, wp2_r, bp2_r, wp3_r, bp3_r,
          wo1_r, bo1_r, wo2_r, bo2_r, wo3_r, bo3_r,
          ws_r, wd_r, bsd_r,
          h_o, nc_o, no_o, fs_o, fd_o):
        agg = (agg_r[0] + agg_r[1])[:, :MSG]
        x_v = x_r[...]
        h1 = jnp.maximum(_dot(x_v, wux_r[...]) + _dot(agg, wua_r[...]) + bu1_r[...], 0.0)
        h = jnp.maximum(_dot(h1, wu2_r[...]) + bu2_r[...], 0.0)
        h_o[...] = jnp.concatenate([h, jnp.zeros((N, LW - CONV), _f32)], axis=1)
        nc = jnp.maximum(_dot(h, wp1_r[...]) + bp1_r[...], 0.0)
        nc = jnp.maximum(_dot(nc, wp2_r[...]) + bp2_r[...], 0.0)
        nc_o[...] = _dot(nc, wp3_r[...]) + bp3_r[...]
        no = jnp.maximum(_dot(h, wo1_r[...]) + bo1_r[...], 0.0)
        no = jnp.maximum(_dot(no, wo2_r[...]) + bo2_r[...], 0.0)
        no_o[...] = _dot(no, wo3_r[...]) + bo3_r[...]
        fs_o[...] = _dot(h, ws_r[...])
        fd_o[...] = _dot(h, wd_r[...]) + bsd_r[...]

    return pl.pallas_call(
        k,
        out_shape=[_sds((N, LW)), _sds((N, NCLS)), _sds((N, ROFF)),
                   _sds((N, CLNK)), _sds((N, CLNK))],
    )(x, agg2, wux, wua, bu1, wu2, bu2, wp1, bp1, wp2, bp2, wp3, bp3,
      wo1, bo1, wo2, bo2, wo3, bo3, ws, wd, bsd)


def _tc_link(fs, fdt, wc, bc, adj):
    R = 256
    grid = (N // R,)

    def k(fs_r, fd_r, wc_r, bc_r, adj_r, out0_r, out1_r):
        fs_v = fs_r[...]
        fd_v = fd_r[...]
        acc0 = jnp.zeros((R, N), _f32)
        acc1 = jnp.zeros((R, N), _f32)
        for kk in range(CLNK):
            t = jnp.maximum(fs_v[:, kk:kk + 1] + fd_v[kk:kk + 1, :], 0.0)
            acc0 = acc0 + t * wc_r[kk, 0]
            acc1 = acc1 + t * wc_r[kk, 1]
        mask = adj_r[...] > 0
        neg = _f32(-1e9)
        out0_r[...] = jnp.where(mask, acc0 + bc_r[0, 0], neg)
        out1_r[...] = jnp.where(mask, acc1 + bc_r[0, 1], neg)

    return pl.pallas_call(
        k,
        grid=grid,
        in_specs=[
            pl.BlockSpec((R, CLNK), lambda i: (i, 0)),
            pl.BlockSpec((CLNK, N), lambda i: (0, 0)),
            pl.BlockSpec(memory_space=pltpu.SMEM),
            pl.BlockSpec(memory_space=pltpu.SMEM),
            pl.BlockSpec((R, N), lambda i: (i, 0)),
        ],
        out_specs=[pl.BlockSpec((R, N), lambda i: (i, 0)),
                   pl.BlockSpec((R, N), lambda i: (i, 0))],
        out_shape=[_sds((N, N)), _sds((N, N))],
    )(fs, fdt, wc, bc, adj)


def _tc_cluster_head(hc3, wq1, bq1, wq2, bq2, wq3, bq3):
    def k(hc_r, wq1_r, bq1_r, wq2_r, bq2_r, wq3_r, bq3_r, oc_o):
        pooled = hc_r[:, 0, :CONV]
        for mm in range(1, M):
            pooled = jnp.maximum(pooled, hc_r[:, mm, :CONV])
        oc = jnp.maximum(_dot(pooled, wq1_r[...]) + bq1_r[...], 0.0)
        oc = jnp.maximum(_dot(oc, wq2_r[...]) + bq2_r[...], 0.0)
        oc_o[...] = _dot(oc, wq3_r[...]) + bq3_r[...]

    return pl.pallas_call(
        k, out_shape=_sds((K, NCLS)),
    )(hc3, wq1, bq1, wq2, bq2, wq3, bq3)


def kernel(node_features, edge_features, edge_index, adj_matrix,
           cluster_node_idx, params):
    p = params
    r1 = lambda b: b.reshape(1, -1)

    wm1 = p["Wm1"]
    x, tu, tv = _tc_node(node_features, p["Wn1"], r1(p["bn1"]),
                         p["Wn2"], r1(p["bn2"]),
                         wm1[:CNODE], wm1[CNODE:2 * CNODE])

    src2 = edge_index[0].reshape(E // CH, CH)
    dst2 = edge_index[1].reshape(E // CH, CH)
    gs, gd = _sc_gather_uv(tu, tv, src2, dst2)

    m2 = _tc_edgemsg(edge_features, gs, gd,
                     p["We1"], r1(p["be1"]), p["We2"], r1(p["be2"]),
                     wm1[2 * CNODE:], r1(p["bm1"]), p["Wm2"], r1(p["bm2"]))

    zer = jnp.zeros((N, LW), _f32)
    agg2 = _sc_scatter_add(m2, dst2, zer)

    wu1 = p["Wu1"]
    h128, node_cls, node_off, fs, fd = _tc_update_heads(
        x, agg2, wu1[:CNODE], wu1[CNODE:], r1(p["bu1"]), p["Wu2"], r1(p["bu2"]),
        p["Wp1"], r1(p["bp1"]), p["Wp2"], r1(p["bp2"]), p["Wp3"], r1(p["bp3"]),
        p["Wo1"], r1(p["bo1"]), p["Wo2"], r1(p["bo2"]), p["Wo3"], r1(p["bo3"]),
        p["Ws"], p["Wd"], r1(p["bsd"]))

    cidx2 = cluster_node_idx.reshape(NW, CH)
    hc = _sc_gather_rows(h128, cidx2)
    obj_cls = _tc_cluster_head(hc.reshape(K, M, LW),
                               p["Wq1"], r1(p["bq1"]), p["Wq2"], r1(p["bq2"]),
                               p["Wq3"], r1(p["bq3"]))

    link0, link1 = _tc_link(fs, fd.T, p["Wc"], p["bc"].reshape(1, -1),
                            adj_matrix)
    link_logits = jnp.stack([link0, link1], axis=-1)

    return (node_cls, node_off, link_logits, obj_cls)


# speedup vs baseline: 1.0123x; 1.0123x over previous
import functools

import jax
import jax.numpy as jnp
from jax import lax
from jax.experimental import pallas as pl
from jax.experimental.pallas import tpu as pltpu
from jax.experimental.pallas import tpu_sc as plsc

N = 2048
E = 131072
D_IN = 64
DE_IN = 16
CNODE = 64
CEDGE = 32
MSG = 64
CONV = 64
CLNK = 16
NCLS = 5
NECLS = 2
ROFF = 2
K = 128
M = 32

LW = 128

NC = 2
NS = 16
NW = NC * NS
CH = 128
G = 512
EW = E // NW

_f32 = jnp.float32


def _sds(shape):
    return jax.ShapeDtypeStruct(shape, _f32)


def _mesh():
    return plsc.VectorSubcoreMesh(core_axis_name="c", subcore_axis_name="s",
                                  num_cores=NC, num_subcores=NS)


def _sc_gather_uv(tu, tv, ei3):
    GG = 256

    @functools.partial(
        pl.kernel,
        out_type=[_sds((E, LW)), _sds((E, LW))],
        mesh=_mesh(),
        scratch_types=[
            pltpu.VMEM((EW // CH, CH), jnp.int32),
            pltpu.VMEM((EW // CH, CH), jnp.int32),
            pltpu.VMEM((GG, LW), _f32),
            pltpu.VMEM((GG, LW), _f32),
            pltpu.SemaphoreType.DMA,
        ],
    )
    def k(tu_hbm, tv_hbm, ei_hbm, gs_hbm, gd_hbm, ibs, ibd, ub, vb, sem):
        wid = lax.axis_index("s") * NC + lax.axis_index("c")
        row0 = wid * (EW // CH)
        pltpu.sync_copy(ei_hbm.at[0, pl.ds(row0, EW // CH)], ibs)
        pltpu.sync_copy(ei_hbm.at[1, pl.ds(row0, EW // CH)], ibd)
        npc = GG // CH

        def body(g, carry):
            off = wid * EW + g * GG
            cps = []
            for j in range(npc):
                sl = pl.ds(j * CH, CH)
                cps.append(pltpu.async_copy(tu_hbm.at[ibs.at[g * npc + j]],
                                            ub.at[sl], sem))
                cps.append(pltpu.async_copy(tv_hbm.at[ibd.at[g * npc + j]],
                                            vb.at[sl], sem))
            for cp in cps:
                cp.wait()
            pltpu.sync_copy(ub, gs_hbm.at[pl.ds(off, GG)])
            pltpu.sync_copy(vb, gd_hbm.at[pl.ds(off, GG)])
            return carry

        lax.fori_loop(0, EW // GG, body, 0)

    return k(tu, tv, ei3)


def _sc_scatter_add(m2, ei3):
    stripe = N // NS

    @functools.partial(
        pl.kernel,
        out_type=_sds((NC, N, LW)),
        mesh=_mesh(),
        scratch_types=[
            pltpu.VMEM((EW // CH, CH), jnp.int32),
            pltpu.VMEM((G, LW), _f32),
            pltpu.VMEM_SHARED((N, LW), _f32),
        ],
    )
    def k(m_hbm, ei_hbm, out_hbm, ib, mb, acc):
        c = lax.axis_index("c")
        s = lax.axis_index("s")

        def zrow(r, carry):
            for j in range(LW // 16):
                mb[r, pl.ds(j * 16, 16)] = jnp.zeros((16,), _f32)
            return carry
        lax.fori_loop(0, stripe, zrow, 0)
        pltpu.sync_copy(mb.at[pl.ds(0, stripe)], acc.at[pl.ds(s * stripe, stripe)])

        wid = s * NC + c
        row0 = wid * (EW // CH)
        pltpu.sync_copy(ei_hbm.at[1, pl.ds(row0, EW // CH)], ib)
        plsc.subcore_barrier()

        def body(g, carry):
            pltpu.sync_copy(m_hbm.at[pl.ds((row0 + g * (G // CH)) * CH, G)], mb)
            for j in range(G // CH):
                pltpu.sync_copy(mb.at[pl.ds(j * CH, CH)],
                                acc.at[ib.at[g * (G // CH) + j]], add=True)
            return carry

        lax.fori_loop(0, EW // G, body, 0)
        plsc.subcore_barrier()
        pltpu.sync_copy(acc.at[pl.ds(s * stripe, stripe)],
                        out_hbm.at[c, pl.ds(s * stripe, stripe)])

    return k(m2, ei3)


def _sc_gather_rows(h128, cidx2):
    @functools.partial(
        pl.kernel,
        out_type=_sds((NW * CH, LW)),
        mesh=_mesh(),
        scratch_types=[
            pltpu.VMEM((CH,), jnp.int32),
            pltpu.VMEM((CH, LW), _f32),
            pltpu.SemaphoreType.DMA,
        ],
    )
    def k(h_hbm, ci_hbm, out_hbm, ib, hb, sem):
        wid = lax.axis_index("s") * NC + lax.axis_index("c")
        pltpu.sync_copy(ci_hbm.at[wid], ib)
        pltpu.async_copy(h_hbm.at[ib], hb, sem).wait()
        pltpu.sync_copy(hb, out_hbm.at[pl.ds(wid * CH, CH)])

    return k(h128, cidx2)


def _dot(a, b):
    return jax.lax.dot(a, b, preferred_element_type=_f32)


def _tc_node(nf, w1, b1, w2, b2, wa, wb):
    def k(nf_r, w1_r, b1_r, w2_r, b2_r, wa_r, wb_r, x_o, tu_o, tv_o):
        x = jnp.maximum(_dot(nf_r[...], w1_r[...]) + b1_r[...], 0.0)
        x = jnp.maximum(_dot(x, w2_r[...]) + b2_r[...], 0.0)
        x_o[...] = x
        z = jnp.zeros((N, LW - MSG), _f32)
        tu_o[...] = jnp.concatenate([_dot(x, wa_r[...]), z], axis=1)
        tv_o[...] = jnp.concatenate([_dot(x, wb_r[...]), z], axis=1)

    return pl.pallas_call(
        k, out_shape=[_sds((N, CNODE)), _sds((N, LW)), _sds((N, LW))],
    )(nf, w1, b1, w2, b2, wa, wb)


def _tc_edgemsg(ef, gs, gd, we1, be1, we2, be2, wc_m, bm1, wm2, bm2):
    BE = 8192
    grid = (E // BE,)

    def k(ef_r, gs_r, gd_r, we1_r, be1_r, we2_r, be2_r, wcm_r, bm1_r,
          wm2_r, bm2_r, m_o):
        e1 = jnp.maximum(_dot(ef_r[...], we1_r[...]) + be1_r[...], 0.0)
        e2 = jnp.maximum(_dot(e1, we2_r[...]) + be2_r[...], 0.0)
        t = _dot(e2, wcm_r[...]) + gs_r[:, :MSG] + gd_r[:, :MSG] + bm1_r[...]
        m1 = jnp.maximum(t, 0.0)
        m = jnp.maximum(_dot(m1, wm2_r[...]) + bm2_r[...], 0.0)
        m_o[...] = jnp.concatenate([m, jnp.zeros((BE, LW - MSG), _f32)], axis=1)

    def full(shape):
        return pl.BlockSpec(shape, lambda i: (0, 0))

    return pl.pallas_call(
        k,
        grid=grid,
        in_specs=[
            pl.BlockSpec((BE, DE_IN), lambda i: (i, 0)),
            pl.BlockSpec((BE, LW), lambda i: (i, 0)),
            pl.BlockSpec((BE, LW), lambda i: (i, 0)),
            full((DE_IN, CEDGE)), full((1, CEDGE)),
            full((CEDGE, CEDGE)), full((1, CEDGE)),
            full((CEDGE, MSG)), full((1, MSG)),
            full((MSG, MSG)), full((1, MSG)),
        ],
        out_specs=pl.BlockSpec((BE, LW), lambda i: (i, 0)),
        out_shape=_sds((E, LW)),
    )(ef, gs, gd, we1, be1, we2, be2, wc_m, bm1, wm2, bm2)


def _tc_update_heads(x, agg2, wux, wua, bu1, wu2, bu2,
                     wp1, bp1, wp2, bp2, wp3, bp3,
                     wo1, bo1, wo2, bo2, wo3, bo3, ws, wd, bsd):
    def k(x_r, agg_r, wux_r, wua_r, bu1_r, wu2_r, bu2_r,
          wp1_r, bp1_r, wp2_r, bp2_r, wp3_r, bp3_r,
          wo1_r, bo1_r, wo2_r, bo2_r, wo3_r, bo3_r,
          ws_r, wd_r, bsd_r,
          h_o, nc_o, no_o, fs_o, fd_o):
        agg = (agg_r[0] + agg_r[1])[:, :MSG]
        x_v = x_r[...]
        h1 = jnp.maximum(_dot(x_v, wux_r[...]) + _dot(agg, wua_r[...]) + bu1_r[...], 0.0)
        h = jnp.maximum(_dot(h1, wu2_r[...]) + bu2_r[...], 0.0)
        h_o[...] = jnp.concatenate([h, jnp.zeros((N, LW - CONV), _f32)], axis=1)
        nc = jnp.maximum(_dot(h, wp1_r[...]) + bp1_r[...], 0.0)
        nc = jnp.maximum(_dot(nc, wp2_r[...]) + bp2_r[...], 0.0)
        nc_o[...] = _dot(nc, wp3_r[...]) + bp3_r[...]
        no = jnp.maximum(_dot(h, wo1_r[...]) + bo1_r[...], 0.0)
        no = jnp.maximum(_dot(no, wo2_r[...]) + bo2_r[...], 0.0)
        no_o[...] = _dot(no, wo3_r[...]) + bo3_r[...]
        fs_o[...] = _dot(h, ws_r[...])
        fd_o[...] = _dot(h, wd_r[...]) + bsd_r[...]

    return pl.pallas_call(
        k,
        out_shape=[_sds((N, LW)), _sds((N, NCLS)), _sds((N, ROFF)),
                   _sds((N, CLNK)), _sds((N, CLNK))],
    )(x, agg2, wux, wua, bu1, wu2, bu2, wp1, bp1, wp2, bp2, wp3, bp3,
      wo1, bo1, wo2, bo2, wo3, bo3, ws, wd, bsd)


def _tc_link(fs, fdt, wc, bc, adj):
    R = 256
    grid = (N // R,)

    def k(fs_r, fd_r, wc_r, bc_r, adj_r, out0_r, out1_r):
        fs_v = fs_r[...]
        fd_v = fd_r[...]
        acc0 = jnp.zeros((R, N), _f32)
        acc1 = jnp.zeros((R, N), _f32)
        for kk in range(CLNK):
            t = jnp.maximum(fs_v[:, kk:kk + 1] + fd_v[kk:kk + 1, :], 0.0)
            acc0 = acc0 + t * wc_r[kk, 0]
            acc1 = acc1 + t * wc_r[kk, 1]
        mask = adj_r[...] > 0
        neg = _f32(-1e9)
        out0_r[...] = jnp.where(mask, acc0 + bc_r[0, 0], neg)
        out1_r[...] = jnp.where(mask, acc1 + bc_r[0, 1], neg)

    return pl.pallas_call(
        k,
        grid=grid,
        in_specs=[
            pl.BlockSpec((R, CLNK), lambda i: (i, 0)),
            pl.BlockSpec((CLNK, N), lambda i: (0, 0)),
            pl.BlockSpec(memory_space=pltpu.SMEM),
            pl.BlockSpec(memory_space=pltpu.SMEM),
            pl.BlockSpec((R, N), lambda i: (i, 0)),
        ],
        out_specs=[pl.BlockSpec((R, N), lambda i: (i, 0)),
                   pl.BlockSpec((R, N), lambda i: (i, 0))],
        out_shape=[_sds((N, N)), _sds((N, N))],
    )(fs, fdt, wc, bc, adj)


def _tc_cluster_head(hc3, wq1, bq1, wq2, bq2, wq3, bq3):
    def k(hc_r, wq1_r, bq1_r, wq2_r, bq2_r, wq3_r, bq3_r, oc_o):
        pooled = hc_r[:, 0, :CONV]
        for mm in range(1, M):
            pooled = jnp.maximum(pooled, hc_r[:, mm, :CONV])
        oc = jnp.maximum(_dot(pooled, wq1_r[...]) + bq1_r[...], 0.0)
        oc = jnp.maximum(_dot(oc, wq2_r[...]) + bq2_r[...], 0.0)
        oc_o[...] = _dot(oc, wq3_r[...]) + bq3_r[...]

    return pl.pallas_call(
        k, out_shape=_sds((K, NCLS)),
    )(hc3, wq1, bq1, wq2, bq2, wq3, bq3)


def kernel(node_features, edge_features, edge_index, adj_matrix,
           cluster_node_idx, params):
    p = params
    r1 = lambda b: b.reshape(1, -1)

    wm1 = p["Wm1"]
    x, tu, tv = _tc_node(node_features, p["Wn1"], r1(p["bn1"]),
                         p["Wn2"], r1(p["bn2"]),
                         wm1[:CNODE], wm1[CNODE:2 * CNODE])

    ei3 = edge_index.reshape(2, E // CH, CH)
    gs, gd = _sc_gather_uv(tu, tv, ei3)

    m2 = _tc_edgemsg(edge_features, gs, gd,
                     p["We1"], r1(p["be1"]), p["We2"], r1(p["be2"]),
                     wm1[2 * CNODE:], r1(p["bm1"]), p["Wm2"], r1(p["bm2"]))

    agg2 = _sc_scatter_add(m2, ei3)

    wu1 = p["Wu1"]
    h128, node_cls, node_off, fs, fd = _tc_update_heads(
        x, agg2, wu1[:CNODE], wu1[CNODE:], r1(p["bu1"]), p["Wu2"], r1(p["bu2"]),
        p["Wp1"], r1(p["bp1"]), p["Wp2"], r1(p["bp2"]), p["Wp3"], r1(p["bp3"]),
        p["Wo1"], r1(p["bo1"]), p["Wo2"], r1(p["bo2"]), p["Wo3"], r1(p["bo3"]),
        p["Ws"], p["Wd"], r1(p["bsd"]))

    cidx2 = cluster_node_idx.reshape(NW, CH)
    hc = _sc_gather_rows(h128, cidx2)
    obj_cls = _tc_cluster_head(hc.reshape(K, M, LW),
                               p["Wq1"], r1(p["bq1"]), p["Wq2"], r1(p["bq2"]),
                               p["Wq3"], r1(p["bq3"]))

    link0, link1 = _tc_link(fs, fd.T, p["Wc"], p["bc"].reshape(1, -1),
                            adj_matrix)
    link_logits = jnp.stack([link0, link1], axis=-1)

    return (node_cls, node_off, link_logits, obj_cls)


# speedup vs baseline: 1.0147x; 1.0024x over previous
import functools

import jax
import jax.numpy as jnp
from jax import lax
from jax.experimental import pallas as pl
from jax.experimental.pallas import tpu as pltpu
from jax.experimental.pallas import tpu_sc as plsc

N = 2048
E = 131072
D_IN = 64
DE_IN = 16
CNODE = 64
CEDGE = 32
MSG = 64
CONV = 64
CLNK = 16
NCLS = 5
NECLS = 2
ROFF = 2
K = 128
M = 32

LW = 128

NC = 2
NS = 16
NW = NC * NS
CH = 128
G = 512
EW = E // NW

_f32 = jnp.float32


def _sds(shape):
    return jax.ShapeDtypeStruct(shape, _f32)


def _mesh():
    return plsc.VectorSubcoreMesh(core_axis_name="c", subcore_axis_name="s",
                                  num_cores=NC, num_subcores=NS)


def _sc_gather_uv(tu, tv, ei3):
    GG = 256

    @functools.partial(
        pl.kernel,
        out_type=[_sds((E, LW)), _sds((E, LW))],
        mesh=_mesh(),
        scratch_types=[
            pltpu.VMEM((EW // CH, CH), jnp.int32),
            pltpu.VMEM((EW // CH, CH), jnp.int32),
            pltpu.VMEM((GG, LW), _f32),
            pltpu.VMEM((GG, LW), _f32),
            pltpu.SemaphoreType.DMA,
            pltpu.SemaphoreType.DMA,
        ],
    )
    def k(tu_hbm, tv_hbm, ei_hbm, gs_hbm, gd_hbm, ibs, ibd, ub, vb, sem, wsem):
        wid = lax.axis_index("s") * NC + lax.axis_index("c")
        row0 = wid * (EW // CH)
        pltpu.sync_copy(ei_hbm.at[0, pl.ds(row0, EW // CH)], ibs)
        pltpu.sync_copy(ei_hbm.at[1, pl.ds(row0, EW // CH)], ibd)
        npc = GG // CH

        def body(g, carry):
            off = wid * EW + g * GG

            @pl.when(g > 0)
            def _():
                pltpu.make_async_copy(ub, gs_hbm.at[pl.ds(off, GG)], wsem).wait()
                pltpu.make_async_copy(vb, gd_hbm.at[pl.ds(off, GG)], wsem).wait()

            cps = []
            for j in range(npc):
                sl = pl.ds(j * CH, CH)
                cps.append(pltpu.async_copy(tu_hbm.at[ibs.at[g * npc + j]],
                                            ub.at[sl], sem))
                cps.append(pltpu.async_copy(tv_hbm.at[ibd.at[g * npc + j]],
                                            vb.at[sl], sem))
            for cp in cps:
                cp.wait()
            pltpu.async_copy(ub, gs_hbm.at[pl.ds(off, GG)], wsem)
            pltpu.async_copy(vb, gd_hbm.at[pl.ds(off, GG)], wsem)
            return carry

        lax.fori_loop(0, EW // GG, body, 0)
        last = wid * EW + (EW // GG - 1) * GG
        pltpu.make_async_copy(ub, gs_hbm.at[pl.ds(last, GG)], wsem).wait()
        pltpu.make_async_copy(vb, gd_hbm.at[pl.ds(last, GG)], wsem).wait()

    return k(tu, tv, ei3)


def _sc_scatter_add(m2, ei3):
    stripe = N // NS

    @functools.partial(
        pl.kernel,
        out_type=_sds((NC, N, LW)),
        mesh=_mesh(),
        scratch_types=[
            pltpu.VMEM((EW // CH, CH), jnp.int32),
            pltpu.VMEM((G, LW), _f32),
            pltpu.VMEM_SHARED((N, LW), _f32),
        ],
    )
    def k(m_hbm, ei_hbm, out_hbm, ib, mb, acc):
        c = lax.axis_index("c")
        s = lax.axis_index("s")

        def zrow(r, carry):
            for j in range(LW // 16):
                mb[r, pl.ds(j * 16, 16)] = jnp.zeros((16,), _f32)
            return carry
        lax.fori_loop(0, stripe, zrow, 0)
        pltpu.sync_copy(mb.at[pl.ds(0, stripe)], acc.at[pl.ds(s * stripe, stripe)])

        wid = s * NC + c
        row0 = wid * (EW // CH)
        pltpu.sync_copy(ei_hbm.at[1, pl.ds(row0, EW // CH)], ib)
        plsc.subcore_barrier()

        def body(g, carry):
            pltpu.sync_copy(m_hbm.at[pl.ds((row0 + g * (G // CH)) * CH, G)], mb)
            for j in range(G // CH):
                pltpu.sync_copy(mb.at[pl.ds(j * CH, CH)],
                                acc.at[ib.at[g * (G // CH) + j]], add=True)
            return carry

        lax.fori_loop(0, EW // G, body, 0)
        plsc.subcore_barrier()
        pltpu.sync_copy(acc.at[pl.ds(s * stripe, stripe)],
                        out_hbm.at[c, pl.ds(s * stripe, stripe)])

    return k(m2, ei3)


def _sc_gather_rows(h128, cidx2):
    @functools.partial(
        pl.kernel,
        out_type=_sds((NW * CH, LW)),
        mesh=_mesh(),
        scratch_types=[
            pltpu.VMEM((CH,), jnp.int32),
            pltpu.VMEM((CH, LW), _f32),
            pltpu.SemaphoreType.DMA,
        ],
    )
    def k(h_hbm, ci_hbm, out_hbm, ib, hb, sem):
        wid = lax.axis_index("s") * NC + lax.axis_index("c")
        pltpu.sync_copy(ci_hbm.at[wid], ib)
        pltpu.async_copy(h_hbm.at[ib], hb, sem).wait()
        pltpu.sync_copy(hb, out_hbm.at[pl.ds(wid * CH, CH)])

    return k(h128, cidx2)


def _dot(a, b):
    return jax.lax.dot(a, b, preferred_element_type=_f32)


def _tc_node(nf, w1, b1, w2, b2, wa, wb):
    def k(nf_r, w1_r, b1_r, w2_r, b2_r, wa_r, wb_r, x_o, tu_o, tv_o):
        x = jnp.maximum(_dot(nf_r[...], w1_r[...]) + b1_r[...], 0.0)
        x = jnp.maximum(_dot(x, w2_r[...]) + b2_r[...], 0.0)
        x_o[...] = x
        z = jnp.zeros((N, LW - MSG), _f32)
        tu_o[...] = jnp.concatenate([_dot(x, wa_r[...]), z], axis=1)
        tv_o[...] = jnp.concatenate([_dot(x, wb_r[...]), z], axis=1)

    return pl.pallas_call(
        k, out_shape=[_sds((N, CNODE)), _sds((N, LW)), _sds((N, LW))],
    )(nf, w1, b1, w2, b2, wa, wb)


def _tc_edgemsg(ef, gs, gd, we1, be1, we2, be2, wc_m, bm1, wm2, bm2):
    BE = 8192
    grid = (E // BE,)

    def k(ef_r, gs_r, gd_r, we1_r, be1_r, we2_r, be2_r, wcm_r, bm1_r,
          wm2_r, bm2_r, m_o):
        e1 = jnp.maximum(_dot(ef_r[...], we1_r[...]) + be1_r[...], 0.0)
        e2 = jnp.maximum(_dot(e1, we2_r[...]) + be2_r[...], 0.0)
        t = _dot(e2, wcm_r[...]) + gs_r[:, :MSG] + gd_r[:, :MSG] + bm1_r[...]
        m1 = jnp.maximum(t, 0.0)
        m = jnp.maximum(_dot(m1, wm2_r[...]) + bm2_r[...], 0.0)
        m_o[...] = jnp.concatenate([m, jnp.zeros((BE, LW - MSG), _f32)], axis=1)

    def full(shape):
        return pl.BlockSpec(shape, lambda i: (0, 0))

    return pl.pallas_call(
        k,
        grid=grid,
        in_specs=[
            pl.BlockSpec((BE, DE_IN), lambda i: (i, 0)),
            pl.BlockSpec((BE, LW), lambda i: (i, 0)),
            pl.BlockSpec((BE, LW), lambda i: (i, 0)),
            full((DE_IN, CEDGE)), full((1, CEDGE)),
            full((CEDGE, CEDGE)), full((1, CEDGE)),
            full((CEDGE, MSG)), full((1, MSG)),
            full((MSG, MSG)), full((1, MSG)),
        ],
        out_specs=pl.BlockSpec((BE, LW), lambda i: (i, 0)),
        out_shape=_sds((E, LW)),
    )(ef, gs, gd, we1, be1, we2, be2, wc_m, bm1, wm2, bm2)


def _tc_update_heads(x, agg2, wux, wua, bu1, wu2, bu2,
                     wp1, bp1, wp2, bp2, wp3, bp3,
                     wo1, bo1, wo2, bo2, wo3, bo3, ws, wd, bsd):
    def k(x_r, agg_r, wux_r, wua_r, bu1_r, wu2_r, bu2_r,
          wp1_r, bp1_r, wp2_r, bp2_r, wp3_r, bp3_r,
          wo1_r, bo1_r, wo2_r, bo2_r, wo3_r, bo3_r,
          ws_r, wd_r, bsd_r,
          h_o, nc_o, no_o, fs_o, fd_o):
        agg = (agg_r[0] + agg_r[1])[:, :MSG]
        x_v = x_r[...]
        h1 = jnp.maximum(_dot(x_v, wux_r[...]) + _dot(agg, wua_r[...]) + bu1_r[...], 0.0)
        h = jnp.maximum(_dot(h1, wu2_r[...]) + bu2_r[...], 0.0)
        h_o[...] = jnp.concatenate([h, jnp.zeros((N, LW - CONV), _f32)], axis=1)
        nc = jnp.maximum(_dot(h, wp1_r[...]) + bp1_r[...], 0.0)
        nc = jnp.maximum(_dot(nc, wp2_r[...]) + bp2_r[...], 0.0)
        nc_o[...] = _dot(nc, wp3_r[...]) + bp3_r[...]
        no = jnp.maximum(_dot(h, wo1_r[...]) + bo1_r[...], 0.0)
        no = jnp.maximum(_dot(no, wo2_r[...]) + bo2_r[...], 0.0)
        no_o[...] = _dot(no, wo3_r[...]) + bo3_r[...]
        fs_o[...] = _dot(h, ws_r[...])
        fd_o[...] = _dot(h, wd_r[...]) + bsd_r[...]

    return pl.pallas_call(
        k,
        out_shape=[_sds((N, LW)), _sds((N, NCLS)), _sds((N, ROFF)),
                   _sds((N, CLNK)), _sds((N, CLNK))],
    )(x, agg2, wux, wua, bu1, wu2, bu2, wp1, bp1, wp2, bp2, wp3, bp3,
      wo1, bo1, wo2, bo2, wo3, bo3, ws, wd, bsd)


def _tc_link(fs, fdt, wc, bc, adj):
    R = 256
    grid = (N // R,)

    def k(fs_r, fd_r, wc_r, bc_r, adj_r, out0_r, out1_r):
        fs_v = fs_r[...]
        fd_v = fd_r[...]
        acc0 = jnp.zeros((R, N), _f32)
        acc1 = jnp.zeros((R, N), _f32)
        for kk in range(CLNK):
            t = jnp.maximum(fs_v[:, kk:kk + 1] + fd_v[kk:kk + 1, :], 0.0)
            acc0 = acc0 + t * wc_r[kk, 0]
            acc1 = acc1 + t * wc_r[kk, 1]
        mask = adj_r[...] > 0
        neg = _f32(-1e9)
        out0_r[...] = jnp.where(mask, acc0 + bc_r[0, 0], neg)
        out1_r[...] = jnp.where(mask, acc1 + bc_r[0, 1], neg)

    return pl.pallas_call(
        k,
        grid=grid,
        in_specs=[
            pl.BlockSpec((R, CLNK), lambda i: (i, 0)),
            pl.BlockSpec((CLNK, N), lambda i: (0, 0)),
            pl.BlockSpec(memory_space=pltpu.SMEM),
            pl.BlockSpec(memory_space=pltpu.SMEM),
            pl.BlockSpec((R, N), lambda i: (i, 0)),
        ],
        out_specs=[pl.BlockSpec((R, N), lambda i: (i, 0)),
                   pl.BlockSpec((R, N), lambda i: (i, 0))],
        out_shape=[_sds((N, N)), _sds((N, N))],
    )(fs, fdt, wc, bc, adj)


def _tc_cluster_head(hc3, wq1, bq1, wq2, bq2, wq3, bq3):
    def k(hc_r, wq1_r, bq1_r, wq2_r, bq2_r, wq3_r, bq3_r, oc_o):
        pooled = hc_r[:, 0, :CONV]
        for mm in range(1, M):
            pooled = jnp.maximum(pooled, hc_r[:, mm, :CONV])
        oc = jnp.maximum(_dot(pooled, wq1_r[...]) + bq1_r[...], 0.0)
        oc = jnp.maximum(_dot(oc, wq2_r[...]) + bq2_r[...], 0.0)
        oc_o[...] = _dot(oc, wq3_r[...]) + bq3_r[...]

    return pl.pallas_call(
        k, out_shape=_sds((K, NCLS)),
    )(hc3, wq1, bq1, wq2, bq2, wq3, bq3)


def kernel(node_features, edge_features, edge_index, adj_matrix,
           cluster_node_idx, params):
    p = params
    r1 = lambda b: b.reshape(1, -1)

    wm1 = p["Wm1"]
    x, tu, tv = _tc_node(node_features, p["Wn1"], r1(p["bn1"]),
                         p["Wn2"], r1(p["bn2"]),
                         wm1[:CNODE], wm1[CNODE:2 * CNODE])

    ei3 = edge_index.reshape(2, E // CH, CH)
    gs, gd = _sc_gather_uv(tu, tv, ei3)

    m2 = _tc_edgemsg(edge_features, gs, gd,
                     p["We1"], r1(p["be1"]), p["We2"], r1(p["be2"]),
                     wm1[2 * CNODE:], r1(p["bm1"]), p["Wm2"], r1(p["bm2"]))

    agg2 = _sc_scatter_add(m2, ei3)

    wu1 = p["Wu1"]
    h128, node_cls, node_off, fs, fd = _tc_update_heads(
        x, agg2, wu1[:CNODE], wu1[CNODE:], r1(p["bu1"]), p["Wu2"], r1(p["bu2"]),
        p["Wp1"], r1(p["bp1"]), p["Wp2"], r1(p["bp2"]), p["Wp3"], r1(p["bp3"]),
        p["Wo1"], r1(p["bo1"]), p["Wo2"], r1(p["bo2"]), p["Wo3"], r1(p["bo3"]),
        p["Ws"], p["Wd"], r1(p["bsd"]))

    cidx2 = cluster_node_idx.reshape(NW, CH)
    hc = _sc_gather_rows(h128, cidx2)
    obj_cls = _tc_cluster_head(hc.reshape(K, M, LW),
                               p["Wq1"], r1(p["bq1"]), p["Wq2"], r1(p["bq2"]),
                               p["Wq3"], r1(p["bq3"]))

    link0, link1 = _tc_link(fs, fd.T, p["Wc"], p["bc"].reshape(1, -1),
                            adj_matrix)
    link_logits = jnp.stack([link0, link1], axis=-1)

    return (node_cls, node_off, link_logits, obj_cls)
